# Initial kernel scaffold; baseline (speedup 1.0000x reference)
import jax
import jax.numpy as jnp
from jax.experimental import pallas as pl


def kernel(x, edge_index, W1, b1, W2, b2, W3, b3):
    raise NotImplementedError("write your pallas kernel here")


# speedup vs baseline: 16.7735x; 16.7735x over previous
import functools

import jax
import jax.numpy as jnp
from jax import lax
from jax.experimental import pallas as pl
from jax.experimental.pallas import tpu as pltpu
from jax.experimental.pallas import tpu_sc as plsc

N = 10000
D = 128
E = 320000
NC = 2
NS = 16
NW = NC * NS
EPW = E // NW
C = 80
CH = EPW // C
RPT = 640
NP = NS * RPT


@functools.cache
def _mesh():
    return plsc.VectorSubcoreMesh(core_axis_name="c", subcore_axis_name="s",
                                  num_cores=NC, num_subcores=NS)


@functools.cache
def _make_sc_degree():
    @functools.partial(
        pl.kernel,
        out_type=jax.ShapeDtypeStruct((NC * NP,), jnp.float32),
        mesh=_mesh(),
        scratch_types=[
            pltpu.VMEM((CH, C), jnp.int32),
            pltpu.VMEM((C,), jnp.float32),
            pltpu.VMEM_SHARED((NP,), jnp.float32),
        ],
    )
    def deg_kernel(dst_hbm, ones_hbm, zeros_hbm, out_hbm, dst_v, ones_v, deg_sp):
        c = lax.axis_index("c")
        s = lax.axis_index("s")
        wid = c * NS + s
        pltpu.sync_copy(dst_hbm.at[wid], dst_v)
        pltpu.sync_copy(ones_hbm, ones_v)
        pltpu.sync_copy(zeros_hbm, deg_sp.at[pl.ds(s * RPT, RPT)])
        plsc.subcore_barrier()

        def body(j, carry):
            pltpu.sync_copy(ones_v, deg_sp.at[dst_v.at[j]], add=True)
            return carry

        lax.fori_loop(0, CH, body, 0)
        plsc.subcore_barrier()
        pltpu.sync_copy(deg_sp.at[pl.ds(s * RPT, RPT)],
                        out_hbm.at[pl.ds(c * NP + s * RPT, RPT)])

    return deg_kernel


@functools.cache
def _make_sc_scatter():
    @functools.partial(
        pl.kernel,
        out_type=jax.ShapeDtypeStruct((NC, NP, D), jnp.float32),
        mesh=_mesh(),
        scratch_types=[
            pltpu.VMEM((CH, C), jnp.int32),
            pltpu.VMEM((CH, C), jnp.int32),
            pltpu.VMEM((C, D), jnp.float32),
            pltpu.VMEM_SHARED((NP, D), jnp.float32),
            pltpu.SemaphoreType.DMA,
        ],
    )
    def scatter_kernel(y_hbm, src_hbm, dst_hbm, zrows_hbm, out_hbm,
                       src_v, dst_v, rows_v, acc_sp, sem):
        c = lax.axis_index("c")
        s = lax.axis_index("s")
        wid = c * NS + s
        base = s * RPT
        pltpu.sync_copy(src_hbm.at[wid], src_v)
        pltpu.sync_copy(dst_hbm.at[wid], dst_v)
        pltpu.sync_copy(zrows_hbm, acc_sp.at[pl.ds(base, RPT)])
        plsc.subcore_barrier()

        def body(j, carry):
            pltpu.async_copy(y_hbm.at[src_v.at[j]], rows_v, sem).wait()
            pltpu.sync_copy(rows_v, acc_sp.at[dst_v.at[j]], add=True)
            return carry

        lax.fori_loop(0, CH, body, 0)
        plsc.subcore_barrier()
        pltpu.sync_copy(acc_sp.at[pl.ds(base, RPT)],
                        out_hbm.at[c, pl.ds(base, RPT)])

    return scatter_kernel


_ROWS = 1000
_GRID = N // _ROWS


def _tc_first_body(deg_ref, x_ref, w_ref, y_ref):
    dinv = lax.rsqrt(deg_ref[...] + 1.0)
    y_ref[...] = dinv * jnp.dot(x_ref[...], w_ref[...],
                                preferred_element_type=jnp.float32)


def _tc_mid_body(deg_ref, s_ref, y_ref, b_ref, w_ref, out_ref):
    dinv = lax.rsqrt(deg_ref[...] + 1.0)
    t = s_ref[0] + s_ref[1] + y_ref[...]
    h = jnp.maximum(dinv * t + b_ref[...], 0.0)
    out_ref[...] = dinv * jnp.dot(h, w_ref[...],
                                  preferred_element_type=jnp.float32)


def _tc_last_body(deg_ref, s_ref, y_ref, b_ref, out_ref):
    dinv = lax.rsqrt(deg_ref[...] + 1.0)
    out_ref[...] = dinv * (s_ref[0] + s_ref[1] + y_ref[...]) + b_ref[...]


_deg_spec = pl.BlockSpec((_ROWS, 1), lambda i: (i, 0))
_row_spec = pl.BlockSpec((_ROWS, D), lambda i: (i, 0))
_s_spec = pl.BlockSpec((NC, _ROWS, D), lambda i: (0, i, 0))
_w_spec = pl.BlockSpec((D, D), lambda i: (0, 0))
_b_spec = pl.BlockSpec((1, D), lambda i: (0, 0))
_out_shape = jax.ShapeDtypeStruct((N, D), jnp.float32)


def _tc_first(deg, x, w):
    return pl.pallas_call(
        _tc_first_body,
        grid=(_GRID,),
        in_specs=[_deg_spec, _row_spec, _w_spec],
        out_specs=_row_spec,
        out_shape=_out_shape,
    )(deg, x, w)


def _tc_mid(deg, s_par, y, b, w):
    return pl.pallas_call(
        _tc_mid_body,
        grid=(_GRID,),
        in_specs=[_deg_spec, _s_spec, _row_spec, _b_spec, _w_spec],
        out_specs=_row_spec,
        out_shape=_out_shape,
    )(deg, s_par, y, b, w)


def _tc_last(deg, s_par, y, b):
    return pl.pallas_call(
        _tc_last_body,
        grid=(_GRID,),
        in_specs=[_deg_spec, _s_spec, _row_spec, _b_spec],
        out_specs=_row_spec,
        out_shape=_out_shape,
    )(deg, s_par, y, b)


def kernel(x, edge_index, W1, b1, W2, b2, W3, b3):
    src = edge_index[0].astype(jnp.int32).reshape(NW, CH, C)
    dst = edge_index[1].astype(jnp.int32).reshape(NW, CH, C)
    ones_c = jnp.ones((C,), jnp.float32)
    zeros_deg = jnp.zeros((RPT,), jnp.float32)
    zrows = jnp.zeros((RPT, D), jnp.float32)
    b1r = b1.reshape(1, D)
    b2r = b2.reshape(1, D)
    b3r = b3.reshape(1, D)

    deg_par = _make_sc_degree()(dst, ones_c, zeros_deg)
    deg = (deg_par[:N] + deg_par[NP:NP + N]).reshape(N, 1)

    y1 = _tc_first(deg, x, W1)
    s1 = _make_sc_scatter()(y1, src, dst, zrows)
    y2 = _tc_mid(deg, s1, y1, b1r, W2)
    s2 = _make_sc_scatter()(y2, src, dst, zrows)
    y3 = _tc_mid(deg, s2, y2, b2r, W3)
    s3 = _make_sc_scatter()(y3, src, dst, zrows)
    return _tc_last(deg, s3, y3, b3r)


# speedup vs baseline: 16.7809x; 1.0004x over previous
import functools

import jax
import jax.numpy as jnp
from jax import lax
from jax.experimental import pallas as pl
from jax.experimental.pallas import tpu as pltpu
from jax.experimental.pallas import tpu_sc as plsc

N = 10000
D = 128
E = 320000
NC = 2
NS = 16
NW = NC * NS
EPW = E // NW
C = 80
CH = EPW // C
RPT = 640
NP = NS * RPT


@functools.cache
def _mesh():
    return plsc.VectorSubcoreMesh(core_axis_name="c", subcore_axis_name="s",
                                  num_cores=NC, num_subcores=NS)


@functools.cache
def _make_sc_degree():
    @functools.partial(
        pl.kernel,
        out_type=jax.ShapeDtypeStruct((NC * NP,), jnp.float32),
        mesh=_mesh(),
        scratch_types=[
            pltpu.VMEM((CH, C), jnp.int32),
            pltpu.VMEM((C,), jnp.float32),
            pltpu.VMEM_SHARED((NP,), jnp.float32),
        ],
    )
    def deg_kernel(dst_hbm, ones_hbm, zeros_hbm, out_hbm, dst_v, ones_v, deg_sp):
        c = lax.axis_index("c")
        s = lax.axis_index("s")
        wid = c * NS + s
        pltpu.sync_copy(dst_hbm.at[wid], dst_v)
        pltpu.sync_copy(ones_hbm, ones_v)
        pltpu.sync_copy(zeros_hbm, deg_sp.at[pl.ds(s * RPT, RPT)])
        plsc.subcore_barrier()

        def body(j, carry):
            pltpu.sync_copy(ones_v, deg_sp.at[dst_v.at[j]], add=True)
            return carry

        lax.fori_loop(0, CH, body, 0)
        plsc.subcore_barrier()
        pltpu.sync_copy(deg_sp.at[pl.ds(s * RPT, RPT)],
                        out_hbm.at[pl.ds(c * NP + s * RPT, RPT)])

    return deg_kernel


@functools.cache
def _make_sc_scatter():
    @functools.partial(
        pl.kernel,
        out_type=jax.ShapeDtypeStruct((NC, NP, D), jnp.float32),
        mesh=_mesh(),
        scratch_types=[
            pltpu.VMEM((CH, C), jnp.int32),
            pltpu.VMEM((CH, C), jnp.int32),
            pltpu.VMEM((C, D), jnp.float32),
            pltpu.VMEM_SHARED((NP, D), jnp.float32),
        ],
    )
    def scatter_kernel(y_hbm, src_hbm, dst_hbm, zrows_hbm, out_hbm,
                       src_v, dst_v, rows, acc_sp):
        c = lax.axis_index("c")
        s = lax.axis_index("s")
        wid = c * NS + s
        base = s * RPT
        pltpu.sync_copy(src_hbm.at[wid], src_v)
        pltpu.sync_copy(dst_hbm.at[wid], dst_v)
        pltpu.sync_copy(zrows_hbm, acc_sp.at[pl.ds(base, RPT)])
        plsc.subcore_barrier()

        def body(j, carry):
            pltpu.sync_copy(y_hbm.at[src_v.at[j]], rows)
            pltpu.sync_copy(rows, acc_sp.at[dst_v.at[j]], add=True)
            return carry

        lax.fori_loop(0, CH, body, 0)

        plsc.subcore_barrier()
        pltpu.sync_copy(acc_sp.at[pl.ds(base, RPT)],
                        out_hbm.at[c, pl.ds(base, RPT)])

    return scatter_kernel


_ROWS = 1000
_GRID = N // _ROWS


def _tc_first_body(deg_ref, x_ref, w_ref, y_ref):
    dinv = lax.rsqrt(deg_ref[...] + 1.0)
    y_ref[...] = dinv * jnp.dot(x_ref[...], w_ref[...],
                                preferred_element_type=jnp.float32)


def _tc_mid_body(deg_ref, s_ref, y_ref, b_ref, w_ref, out_ref):
    dinv = lax.rsqrt(deg_ref[...] + 1.0)
    t = s_ref[0] + s_ref[1] + y_ref[...]
    h = jnp.maximum(dinv * t + b_ref[...], 0.0)
    out_ref[...] = dinv * jnp.dot(h, w_ref[...],
                                  preferred_element_type=jnp.float32)


def _tc_last_body(deg_ref, s_ref, y_ref, b_ref, out_ref):
    dinv = lax.rsqrt(deg_ref[...] + 1.0)
    out_ref[...] = dinv * (s_ref[0] + s_ref[1] + y_ref[...]) + b_ref[...]


_deg_spec = pl.BlockSpec((_ROWS, 1), lambda i: (i, 0))
_row_spec = pl.BlockSpec((_ROWS, D), lambda i: (i, 0))
_s_spec = pl.BlockSpec((NC, _ROWS, D), lambda i: (0, i, 0))
_w_spec = pl.BlockSpec((D, D), lambda i: (0, 0))
_b_spec = pl.BlockSpec((1, D), lambda i: (0, 0))
_out_shape = jax.ShapeDtypeStruct((N, D), jnp.float32)


def _tc_first(deg, x, w):
    return pl.pallas_call(
        _tc_first_body,
        grid=(_GRID,),
        in_specs=[_deg_spec, _row_spec, _w_spec],
        out_specs=_row_spec,
        out_shape=_out_shape,
    )(deg, x, w)


def _tc_mid(deg, s_par, y, b, w):
    return pl.pallas_call(
        _tc_mid_body,
        grid=(_GRID,),
        in_specs=[_deg_spec, _s_spec, _row_spec, _b_spec, _w_spec],
        out_specs=_row_spec,
        out_shape=_out_shape,
    )(deg, s_par, y, b, w)


def _tc_last(deg, s_par, y, b):
    return pl.pallas_call(
        _tc_last_body,
        grid=(_GRID,),
        in_specs=[_deg_spec, _s_spec, _row_spec, _b_spec],
        out_specs=_row_spec,
        out_shape=_out_shape,
    )(deg, s_par, y, b)


def kernel(x, edge_index, W1, b1, W2, b2, W3, b3):
    src = edge_index[0].astype(jnp.int32).reshape(NW, CH, C)
    dst = edge_index[1].astype(jnp.int32).reshape(NW, CH, C)
    ones_c = jnp.ones((C,), jnp.float32)
    zeros_deg = jnp.zeros((RPT,), jnp.float32)
    zrows = jnp.zeros((RPT, D), jnp.float32)
    b1r = b1.reshape(1, D)
    b2r = b2.reshape(1, D)
    b3r = b3.reshape(1, D)

    deg_par = _make_sc_degree()(dst, ones_c, zeros_deg)
    deg = (deg_par[:N] + deg_par[NP:NP + N]).reshape(N, 1)

    y1 = _tc_first(deg, x, W1)
    s1 = _make_sc_scatter()(y1, src, dst, zrows)
    y2 = _tc_mid(deg, s1, y1, b1r, W2)
    s2 = _make_sc_scatter()(y2, src, dst, zrows)
    y3 = _tc_mid(deg, s2, y2, b2r, W3)
    s3 = _make_sc_scatter()(y3, src, dst, zrows)
    return _tc_last(deg, s3, y3, b3r)
